# Initial kernel scaffold; baseline (speedup 1.0000x reference)
import jax
import jax.numpy as jnp
from jax.experimental import pallas as pl


def kernel(x, edge_index, edge_weights, W_gat, a_src, a_dst, W_emb, b_emb):
    raise NotImplementedError("write your pallas kernel here")


# speedup vs baseline: 32.8764x; 32.8764x over previous
import functools

import jax
import jax.numpy as jnp
from jax import lax
from jax.experimental import pallas as pl
from jax.experimental.pallas import tpu as pltpu
from jax.experimental.pallas import tpu_sc as plsc

N = 50000
E = 800000
H = 4
C = 16
D = H * C
EMB = 64
NEG_SLOPE = 0.1

NC = 2
NS = 16
NW = NC * NS

BLK = 128
BLOCKS_PER_W = 200
E_PAD = NW * BLOCKS_PER_W * BLK
NT = 50560
ZROWS = NT // NS
OROWS = N // NS

NBLK = 512
TC_GRID = (N + NBLK - 1) // NBLK


def _prologue_body(x_ref, wg_ref, ad_ref, h_ref, alpha_ref):
    h = jnp.dot(x_ref[...], wg_ref[...], preferred_element_type=jnp.float32)
    hh = h.reshape(NBLK, H, C).transpose(1, 0, 2)
    h_ref[...] = hh
    alpha_ref[...] = jnp.sum(hh * ad_ref[...][:, None, :], axis=-1)


def _epilogue_body(num_ref, den_ref, we_ref, be_ref, out_ref):
    num = num_ref[0] + num_ref[1]
    den = jnp.sum(den_ref[...], axis=1) + 1e-16
    o = (num / den[:, :, None]).transpose(1, 0, 2).reshape(NBLK, D)
    o = jnp.where(o > 0, o, jnp.exp(o) - 1.0)
    out_ref[...] = (
        jnp.dot(o, we_ref[...], preferred_element_type=jnp.float32)
        + be_ref[...]
    )


def _sc_body(h_ref, alpha_ref, asrc_ref, src_ref, dst_ref, w_ref, z_ref,
             z1_ref, num_ref, den_ref,
             tab_v, rows_v, src_v, dst_v, w_v, p_v, arow_v,
             num_sp, den_sp, gsem, ssem):
    cid = lax.axis_index("c")
    sid = lax.axis_index("s")
    wid = cid * NS + sid
    ebase = wid * (BLOCKS_PER_W * BLK)
    iota = lax.iota(jnp.int32, 16)
    zero16 = jnp.zeros((16,), jnp.float32)

    for k in range(H):
        pltpu.sync_copy(alpha_ref.at[pl.ds(k * NT, NT)], tab_v)
        pltpu.sync_copy(asrc_ref.at[pl.ds(k * C, C)], arow_v)
        pltpu.sync_copy(z_ref, num_sp.at[pl.ds(sid * ZROWS, ZROWS)])
        pltpu.sync_copy(z1_ref, den_sp.at[pl.ds(sid * (NT // NS), NT // NS)])

        arow16 = arow_v[...]
        a_splat = [jnp.full((16,), arow16[c]) for c in range(C)]
        plsc.subcore_barrier()

        def _block(b, _):
            base = ebase + b * BLK
            pltpu.sync_copy(src_ref.at[pl.ds(base, BLK)], src_v)
            pltpu.sync_copy(dst_ref.at[pl.ds(base, BLK)], dst_v)
            pltpu.sync_copy(w_ref.at[pl.ds(base, BLK)], w_v)
            pltpu.async_copy(h_ref.at[k].at[src_v], rows_v, gsem).wait()
            for g in range(BLK // 16):
                ridx = iota + (g * 16)
                cols = []
                alpha = zero16
                for c in range(C):
                    col = plsc.load_gather(
                        rows_v, [ridx, jnp.full((16,), c, jnp.int32)])
                    cols.append(col)
                    alpha = alpha + col * a_splat[c]
                dst16 = dst_v[pl.ds(g * 16, 16)]
                w16 = w_v[pl.ds(g * 16, 16)]
                ad = plsc.load_gather(tab_v, [dst16])
                e = alpha + ad
                e = jnp.where(e >= 0.0, e, e * NEG_SLOPE) * w16
                p = jnp.exp(e)
                p_v[pl.ds(g * 16, 16)] = p
                for c in range(C):
                    plsc.store_scatter(
                        rows_v, [ridx, jnp.full((16,), c, jnp.int32)],
                        cols[c] * p)
            pltpu.sync_copy(rows_v, num_sp.at[dst_v], add=True)
            pltpu.sync_copy(p_v, den_sp.at[dst_v], add=True)
            return 0
        lax.fori_loop(0, BLOCKS_PER_W, _block, 0)

        plsc.subcore_barrier()
        pltpu.sync_copy(num_sp.at[pl.ds(sid * ZROWS, ZROWS)],
                        num_ref.at[cid, k, pl.ds(sid * ZROWS, ZROWS)])
        dbase = pl.multiple_of((k * NC + cid) * NT + sid * (NT // NS), 8)
        pltpu.sync_copy(den_sp.at[pl.ds(sid * (NT // NS), NT // NS)],
                        den_ref.at[pl.ds(dbase, NT // NS)])
        plsc.subcore_barrier()


_sc_call = pl.kernel(
    _sc_body,
    out_type=(
        jax.ShapeDtypeStruct((NC, H, NT, C), jnp.float32),
        jax.ShapeDtypeStruct((H * NC * NT,), jnp.float32),
    ),
    mesh=plsc.VectorSubcoreMesh(core_axis_name="c", subcore_axis_name="s",
                                num_cores=NC, num_subcores=NS),
    compiler_params=pltpu.CompilerParams(needs_layout_passes=False,
                                         use_tc_tiling_on_sc=False),
    scratch_types=(
        pltpu.VMEM((NT,), jnp.float32),
        pltpu.VMEM((BLK, C), jnp.float32),
        pltpu.VMEM((BLK,), jnp.int32),
        pltpu.VMEM((BLK,), jnp.int32),
        pltpu.VMEM((BLK,), jnp.float32),
        pltpu.VMEM((BLK,), jnp.float32),
        pltpu.VMEM((C,), jnp.float32),
        pltpu.VMEM_SHARED((NT, C), jnp.float32),
        pltpu.VMEM_SHARED((NT,), jnp.float32),
        pltpu.SemaphoreType.DMA,
        pltpu.SemaphoreType.DMA,
    ),
)


@functools.partial(jax.jit, static_argnums=())
def kernel(x, edge_index, edge_weights, W_gat, a_src, a_dst, W_emb, b_emb):
    h_heads, alpha_dst = pl.pallas_call(
        _prologue_body,
        grid=(TC_GRID,),
        in_specs=[
            pl.BlockSpec((NBLK, D), lambda i: (i, 0)),
            pl.BlockSpec((D, D), lambda i: (0, 0)),
            pl.BlockSpec((H, C), lambda i: (0, 0)),
        ],
        out_specs=[
            pl.BlockSpec((H, NBLK, C), lambda i: (0, i, 0)),
            pl.BlockSpec((H, NBLK), lambda i: (0, i)),
        ],
        out_shape=[
            jax.ShapeDtypeStruct((H, N, C), jnp.float32),
            jax.ShapeDtypeStruct((H, N), jnp.float32),
        ],
    )(x, W_gat, a_dst)

    alpha_pad = jnp.pad(alpha_dst, ((0, 0), (0, NT - N))).reshape(-1)
    npad = E_PAD - E
    pad_i = lax.iota(jnp.int32, npad)
    src_p = jnp.concatenate([edge_index[0], pad_i % 4096])
    dst_p = jnp.concatenate([edge_index[1], N + pad_i % 512])
    w_p = jnp.concatenate([edge_weights, jnp.zeros((npad,), jnp.float32)])
    zeros = jnp.zeros((ZROWS, C), jnp.float32)
    zeros1 = jnp.zeros((NT // NS,), jnp.float32)

    num, den = _sc_call(h_heads, alpha_pad, a_src.reshape(-1), src_p,
                        dst_p, w_p, zeros, zeros1)

    den4 = den.reshape(H, NC, NT)
    emb = pl.pallas_call(
        _epilogue_body,
        grid=(TC_GRID,),
        in_specs=[
            pl.BlockSpec((NC, H, NBLK, C), lambda i: (0, 0, i, 0)),
            pl.BlockSpec((H, NC, NBLK), lambda i: (0, 0, i)),
            pl.BlockSpec((D, EMB), lambda i: (0, 0)),
            pl.BlockSpec((1, EMB), lambda i: (0, 0)),
        ],
        out_specs=pl.BlockSpec((NBLK, EMB), lambda i: (i, 0)),
        out_shape=jax.ShapeDtypeStruct((N, EMB), jnp.float32),
    )(num, den4, W_emb, b_emb.reshape(1, EMB))
    return emb


# speedup vs baseline: 42.6686x; 1.2978x over previous
import functools

import jax
import jax.numpy as jnp
from jax import lax
from jax.experimental import pallas as pl
from jax.experimental.pallas import tpu as pltpu
from jax.experimental.pallas import tpu_sc as plsc

N = 50000
E = 800000
H = 4
C = 16
D = H * C
EMB = 64
NEG_SLOPE = 0.1

NC = 2
NS = 16
NW = NC * NS

BLK = 128
BLOCKS_PER_W = 200
E_PAD = NW * BLOCKS_PER_W * BLK
NT = 50560
ZROWS = NT // NS
OROWS = N // NS

NBLK = 512
TC_GRID = (N + NBLK - 1) // NBLK


def _prologue_body(x_ref, wg_ref, ad_ref, h_ref, alpha_ref):
    h = jnp.dot(x_ref[...], wg_ref[...], preferred_element_type=jnp.float32)
    hh = h.reshape(NBLK, H, C).transpose(1, 0, 2)
    h_ref[...] = hh
    alpha_ref[...] = jnp.sum(hh * ad_ref[...][:, None, :], axis=-1)


def _epilogue_body(num_ref, den_ref, we_ref, be_ref, out_ref):
    num = num_ref[0] + num_ref[1]
    den = jnp.sum(den_ref[...], axis=1) + 1e-16
    o = (num / den[:, :, None]).transpose(1, 0, 2).reshape(NBLK, D)
    o = jnp.where(o > 0, o, jnp.exp(o) - 1.0)
    out_ref[...] = (
        jnp.dot(o, we_ref[...], preferred_element_type=jnp.float32)
        + be_ref[...]
    )


def _sc_body(h_ref, alpha_ref, asrc_ref, src_ref, dst_ref, w_ref, z_ref,
             z1_ref, num_ref, den_ref,
             tab_v, arow_v, rows_v, src_v, dst_v, w_v, p_v,
             num_sp, den_sp, gsem, nsem, dsem):
    cid = lax.axis_index("c")
    sid = lax.axis_index("s")
    wid = cid * NS + sid
    ebase = wid * (BLOCKS_PER_W * BLK)
    iota = lax.iota(jnp.int32, 16)
    zero16 = jnp.zeros((16,), jnp.float32)

    def _head(k, _):
        pltpu.sync_copy(alpha_ref.at[pl.ds(pl.multiple_of(k * NT, 8), NT)],
                        tab_v)
        pltpu.sync_copy(asrc_ref.at[pl.ds(pl.multiple_of(k * C, 8), C)],
                        arow_v)
        pltpu.sync_copy(z_ref, num_sp.at[pl.ds(sid * ZROWS, ZROWS)])
        pltpu.sync_copy(z1_ref, den_sp.at[pl.ds(sid * (NT // NS), NT // NS)])

        arow16 = arow_v[...]
        a_splat = [jnp.full((16,), arow16[c]) for c in range(C)]
        plsc.subcore_barrier()

        def _fetch(b, j):
            base = ebase + b * BLK
            pltpu.sync_copy(src_ref.at[pl.ds(base, BLK)], src_v[j])
            pltpu.sync_copy(dst_ref.at[pl.ds(base, BLK)], dst_v[j])
            pltpu.sync_copy(w_ref.at[pl.ds(base, BLK)], w_v[j])
            pltpu.async_copy(h_ref.at[k].at[src_v[j]], rows_v[j], gsem[j])

        def _wait_scatters(j):
            pltpu.make_async_copy(rows_v[j], num_sp.at[dst_v[j]],
                                  nsem[j]).wait()
            pltpu.make_async_copy(p_v[j], den_sp.at[dst_v[j]],
                                  dsem[j]).wait()

        def _compute(j):
            pltpu.make_async_copy(h_ref.at[k].at[src_v[j]], rows_v[j],
                                  gsem[j]).wait()
            for g in range(BLK // 16):
                ridx = iota + (g * 16)
                cols = []
                alpha = zero16
                for c in range(C):
                    col = plsc.load_gather(
                        rows_v[j], [ridx, jnp.full((16,), c, jnp.int32)])
                    cols.append(col)
                    alpha = alpha + col * a_splat[c]
                dst16 = dst_v[j][pl.ds(g * 16, 16)]
                w16 = w_v[j][pl.ds(g * 16, 16)]
                ad = plsc.load_gather(tab_v, [dst16])
                e = alpha + ad
                e = jnp.where(e >= 0.0, e, e * NEG_SLOPE) * w16
                p = jnp.exp(e)
                p_v[j][pl.ds(g * 16, 16)] = p
                for c in range(C):
                    plsc.store_scatter(
                        rows_v[j], [ridx, jnp.full((16,), c, jnp.int32)],
                        cols[c] * p)
            pltpu.async_copy(rows_v[j], num_sp.at[dst_v[j]], nsem[j],
                             add=True)
            pltpu.async_copy(p_v[j], den_sp.at[dst_v[j]], dsem[j], add=True)

        _fetch(0, 0)
        _fetch(1, 1)
        _fetch(2, 2)
        _compute(0)
        _compute(1)

        def _trio(t, _):
            for j, off in ((2, 2), (0, 3), (1, 4)):
                b = 3 * t + off
                nj = (j + 1) % 3
                _wait_scatters(nj)

                @pl.when(b + 1 < BLOCKS_PER_W)
                def _():
                    _fetch(b + 1, nj)
                _compute(j)
            return 0

        lax.fori_loop(0, (BLOCKS_PER_W - 2) // 3, _trio, 0)
        _wait_scatters(0)
        _wait_scatters(1)

        plsc.subcore_barrier()
        pltpu.sync_copy(num_sp.at[pl.ds(sid * ZROWS, ZROWS)],
                        num_ref.at[cid, k, pl.ds(sid * ZROWS, ZROWS)])
        dbase = pl.multiple_of((k * NC + cid) * NT + sid * (NT // NS), 8)
        pltpu.sync_copy(den_sp.at[pl.ds(sid * (NT // NS), NT // NS)],
                        den_ref.at[pl.ds(dbase, NT // NS)])
        plsc.subcore_barrier()
        return 0

    lax.fori_loop(0, H, _head, 0)


_sc_call = pl.kernel(
    _sc_body,
    out_type=(
        jax.ShapeDtypeStruct((NC, H, NT, C), jnp.float32),
        jax.ShapeDtypeStruct((H * NC * NT,), jnp.float32),
    ),
    mesh=plsc.VectorSubcoreMesh(core_axis_name="c", subcore_axis_name="s",
                                num_cores=NC, num_subcores=NS),
    compiler_params=pltpu.CompilerParams(needs_layout_passes=False,
                                         use_tc_tiling_on_sc=False),
    scratch_types=(
        pltpu.VMEM((NT,), jnp.float32),
        pltpu.VMEM((C,), jnp.float32),
        [pltpu.VMEM((BLK, C), jnp.float32)] * 3,
        [pltpu.VMEM((BLK,), jnp.int32)] * 3,
        [pltpu.VMEM((BLK,), jnp.int32)] * 3,
        [pltpu.VMEM((BLK,), jnp.float32)] * 3,
        [pltpu.VMEM((BLK,), jnp.float32)] * 3,
        pltpu.VMEM_SHARED((NT, C), jnp.float32),
        pltpu.VMEM_SHARED((NT,), jnp.float32),
        [pltpu.SemaphoreType.DMA] * 3,
        [pltpu.SemaphoreType.DMA] * 3,
        [pltpu.SemaphoreType.DMA] * 3,
    ),
)


@functools.partial(jax.jit, static_argnums=())
def kernel(x, edge_index, edge_weights, W_gat, a_src, a_dst, W_emb, b_emb):
    h_heads, alpha_dst = pl.pallas_call(
        _prologue_body,
        grid=(TC_GRID,),
        in_specs=[
            pl.BlockSpec((NBLK, D), lambda i: (i, 0)),
            pl.BlockSpec((D, D), lambda i: (0, 0)),
            pl.BlockSpec((H, C), lambda i: (0, 0)),
        ],
        out_specs=[
            pl.BlockSpec((H, NBLK, C), lambda i: (0, i, 0)),
            pl.BlockSpec((H, NBLK), lambda i: (0, i)),
        ],
        out_shape=[
            jax.ShapeDtypeStruct((H, N, C), jnp.float32),
            jax.ShapeDtypeStruct((H, N), jnp.float32),
        ],
    )(x, W_gat, a_dst)

    alpha_pad = jnp.pad(alpha_dst, ((0, 0), (0, NT - N))).reshape(-1)
    npad = E_PAD - E
    pad_i = lax.iota(jnp.int32, npad)
    src_p = jnp.concatenate([edge_index[0], pad_i % 4096])
    dst_p = jnp.concatenate([edge_index[1], N + pad_i % 512])
    w_p = jnp.concatenate([edge_weights, jnp.zeros((npad,), jnp.float32)])
    zeros = jnp.zeros((ZROWS, C), jnp.float32)
    zeros1 = jnp.zeros((NT // NS,), jnp.float32)

    num, den = _sc_call(h_heads, alpha_pad, a_src.reshape(-1), src_p,
                        dst_p, w_p, zeros, zeros1)

    den4 = den.reshape(H, NC, NT)
    emb = pl.pallas_call(
        _epilogue_body,
        grid=(TC_GRID,),
        in_specs=[
            pl.BlockSpec((NC, H, NBLK, C), lambda i: (0, 0, i, 0)),
            pl.BlockSpec((H, NC, NBLK), lambda i: (0, 0, i)),
            pl.BlockSpec((D, EMB), lambda i: (0, 0)),
            pl.BlockSpec((1, EMB), lambda i: (0, 0)),
        ],
        out_specs=pl.BlockSpec((NBLK, EMB), lambda i: (i, 0)),
        out_shape=jax.ShapeDtypeStruct((N, EMB), jnp.float32),
    )(num, den4, W_emb, b_emb.reshape(1, EMB))
    return emb


# speedup vs baseline: 57.3871x; 1.3449x over previous
import functools

import jax
import jax.numpy as jnp
from jax import lax
from jax.experimental import pallas as pl
from jax.experimental.pallas import tpu as pltpu
from jax.experimental.pallas import tpu_sc as plsc

N = 50000
E = 800000
H = 4
C = 16
D = H * C
EMB = 64
NEG_SLOPE = 0.1

NC = 2
NS = 16
NW = NC * NS

BLK = 128
BLOCKS_PER_W = 200
E_PAD = NW * BLOCKS_PER_W * BLK
NT = 50560
ZROWS = NT // NS
OROWS = N // NS

NBLK = 512
TC_GRID = (N + NBLK - 1) // NBLK


def _prologue_body(x_ref, wg_ref, ad_ref, h_ref, alpha_ref):
    h = jnp.dot(x_ref[...], wg_ref[...], preferred_element_type=jnp.float32)
    hh = h.reshape(NBLK, H, C).transpose(1, 0, 2)
    h_ref[...] = hh
    alpha_ref[...] = jnp.sum(hh * ad_ref[...][:, None, :], axis=-1)


def _epilogue_body(num_ref, den_ref, we_ref, be_ref, out_ref):
    num = num_ref[0] + num_ref[1]
    den = jnp.sum(den_ref[...], axis=1) + 1e-16
    o = (num / den[:, :, None]).transpose(1, 0, 2).reshape(NBLK, D)
    o = jnp.where(o > 0, o, jnp.exp(o) - 1.0)
    out_ref[...] = (
        jnp.dot(o, we_ref[...], preferred_element_type=jnp.float32)
        + be_ref[...]
    )


def _sc_body(h_ref, alpha_ref, asrc_ref, edata_ref, z_ref,
             z1_ref, num_ref, den_ref,
             tab_v, arow_v, rows_v, ebuf_v, p_v,
             num_sp, den_sp, gsem, nsem, dsem):
    cid = lax.axis_index("c")
    sid = lax.axis_index("s")
    wid = cid * NS + sid
    bbase = wid * BLOCKS_PER_W
    iota = lax.iota(jnp.int32, 16)
    zero16 = jnp.zeros((16,), jnp.float32)

    def _head(k, _):
        pltpu.sync_copy(alpha_ref.at[pl.ds(pl.multiple_of(k * NT, 8), NT)],
                        tab_v)
        pltpu.sync_copy(asrc_ref.at[pl.ds(pl.multiple_of(k * C, 8), C)],
                        arow_v)
        pltpu.sync_copy(z_ref, num_sp.at[pl.ds(sid * ZROWS, ZROWS)])
        pltpu.sync_copy(z1_ref, den_sp.at[pl.ds(sid * (NT // NS), NT // NS)])

        arow16 = arow_v[...]
        a_splat = [jnp.full((16,), arow16[c]) for c in range(C)]
        plsc.subcore_barrier()

        def _fetch(b, j):
            pltpu.sync_copy(edata_ref.at[bbase + b], ebuf_v[j])
            pltpu.async_copy(h_ref.at[k].at[ebuf_v[j].at[0]], rows_v[j],
                             gsem[j])

        def _wait_scatters(j):
            pltpu.make_async_copy(rows_v[j], num_sp.at[ebuf_v[j].at[1]],
                                  nsem[j]).wait()
            pltpu.make_async_copy(p_v[j], den_sp.at[ebuf_v[j].at[1]],
                                  dsem[j]).wait()

        def _compute(j):
            pltpu.make_async_copy(h_ref.at[k].at[ebuf_v[j].at[0]], rows_v[j],
                                  gsem[j]).wait()
            for g in range(BLK // 16):
                ridx = iota + (g * 16)
                cols = []
                alpha = zero16
                for c in range(C):
                    col = plsc.load_gather(
                        rows_v[j], [ridx, jnp.full((16,), c, jnp.int32)])
                    cols.append(col)
                    alpha = alpha + col * a_splat[c]
                dst16 = ebuf_v[j][1, pl.ds(g * 16, 16)]
                w16 = plsc.bitcast(ebuf_v[j][2, pl.ds(g * 16, 16)],
                                   jnp.float32)
                ad = plsc.load_gather(tab_v, [dst16])
                e = alpha + ad
                e = jnp.where(e >= 0.0, e, e * NEG_SLOPE) * w16
                p = jnp.exp(e)
                p_v[j][pl.ds(g * 16, 16)] = p
                for c in range(C):
                    plsc.store_scatter(
                        rows_v[j], [ridx, jnp.full((16,), c, jnp.int32)],
                        cols[c] * p)
            pltpu.async_copy(rows_v[j], num_sp.at[ebuf_v[j].at[1]], nsem[j],
                             add=True)
            pltpu.async_copy(p_v[j], den_sp.at[ebuf_v[j].at[1]], dsem[j],
                             add=True)

        _fetch(0, 0)
        _fetch(1, 1)
        _fetch(2, 2)
        _compute(0)
        _compute(1)

        def _trio(t, _):
            for j, off in ((2, 2), (0, 3), (1, 4)):
                b = 3 * t + off
                nj = (j + 1) % 3
                _wait_scatters(nj)

                @pl.when(b + 1 < BLOCKS_PER_W)
                def _():
                    _fetch(b + 1, nj)
                _compute(j)
            return 0

        lax.fori_loop(0, (BLOCKS_PER_W - 2) // 3, _trio, 0)
        _wait_scatters(0)
        _wait_scatters(1)

        plsc.subcore_barrier()
        pltpu.sync_copy(num_sp.at[pl.ds(sid * ZROWS, ZROWS)],
                        num_ref.at[cid, k, pl.ds(sid * ZROWS, ZROWS)])
        dbase = pl.multiple_of((k * NC + cid) * NT + sid * (NT // NS), 8)
        pltpu.sync_copy(den_sp.at[pl.ds(sid * (NT // NS), NT // NS)],
                        den_ref.at[pl.ds(dbase, NT // NS)])
        plsc.subcore_barrier()
        return 0

    lax.fori_loop(0, H, _head, 0)


_sc_call = pl.kernel(
    _sc_body,
    out_type=(
        jax.ShapeDtypeStruct((NC, H, NT, C), jnp.float32),
        jax.ShapeDtypeStruct((H * NC * NT,), jnp.float32),
    ),
    mesh=plsc.VectorSubcoreMesh(core_axis_name="c", subcore_axis_name="s",
                                num_cores=NC, num_subcores=NS),
    compiler_params=pltpu.CompilerParams(needs_layout_passes=False,
                                         use_tc_tiling_on_sc=False),
    scratch_types=(
        pltpu.VMEM((NT,), jnp.float32),
        pltpu.VMEM((C,), jnp.float32),
        [pltpu.VMEM((BLK, C), jnp.float32)] * 3,
        [pltpu.VMEM((3, BLK), jnp.int32)] * 3,
        [pltpu.VMEM((BLK,), jnp.float32)] * 3,
        pltpu.VMEM_SHARED((NT, C), jnp.float32),
        pltpu.VMEM_SHARED((NT,), jnp.float32),
        [pltpu.SemaphoreType.DMA] * 3,
        [pltpu.SemaphoreType.DMA] * 3,
        [pltpu.SemaphoreType.DMA] * 3,
    ),
)


@functools.partial(jax.jit, static_argnums=())
def kernel(x, edge_index, edge_weights, W_gat, a_src, a_dst, W_emb, b_emb):
    h_heads, alpha_dst = pl.pallas_call(
        _prologue_body,
        grid=(TC_GRID,),
        in_specs=[
            pl.BlockSpec((NBLK, D), lambda i: (i, 0)),
            pl.BlockSpec((D, D), lambda i: (0, 0)),
            pl.BlockSpec((H, C), lambda i: (0, 0)),
        ],
        out_specs=[
            pl.BlockSpec((H, NBLK, C), lambda i: (0, i, 0)),
            pl.BlockSpec((H, NBLK), lambda i: (0, i)),
        ],
        out_shape=[
            jax.ShapeDtypeStruct((H, N, C), jnp.float32),
            jax.ShapeDtypeStruct((H, N), jnp.float32),
        ],
    )(x, W_gat, a_dst)

    alpha_pad = jnp.pad(alpha_dst, ((0, 0), (0, NT - N))).reshape(-1)
    npad = E_PAD - E
    pad_i = lax.iota(jnp.int32, npad)
    src_p = jnp.concatenate([edge_index[0], pad_i % 4096])
    dst_p = jnp.concatenate([edge_index[1], N + pad_i % 512])
    w_p = jnp.concatenate([edge_weights, jnp.zeros((npad,), jnp.float32)])
    edata = jnp.stack([src_p.reshape(-1, BLK), dst_p.reshape(-1, BLK),
                       lax.bitcast_convert_type(w_p, jnp.int32)
                       .reshape(-1, BLK)], axis=1)
    zeros = jnp.zeros((ZROWS, C), jnp.float32)
    zeros1 = jnp.zeros((NT // NS,), jnp.float32)

    num, den = _sc_call(h_heads, alpha_pad, a_src.reshape(-1), edata,
                        zeros, zeros1)

    den4 = den.reshape(H, NC, NT)
    emb = pl.pallas_call(
        _epilogue_body,
        grid=(TC_GRID,),
        in_specs=[
            pl.BlockSpec((NC, H, NBLK, C), lambda i: (0, 0, i, 0)),
            pl.BlockSpec((H, NC, NBLK), lambda i: (0, 0, i)),
            pl.BlockSpec((D, EMB), lambda i: (0, 0)),
            pl.BlockSpec((1, EMB), lambda i: (0, 0)),
        ],
        out_specs=pl.BlockSpec((NBLK, EMB), lambda i: (i, 0)),
        out_shape=jax.ShapeDtypeStruct((N, EMB), jnp.float32),
    )(num, den4, W_emb, b_emb.reshape(1, EMB))
    return emb


# speedup vs baseline: 70.1226x; 1.2219x over previous
import functools

import jax
import jax.numpy as jnp
from jax import lax
from jax.experimental import pallas as pl
from jax.experimental.pallas import tpu as pltpu
from jax.experimental.pallas import tpu_sc as plsc

N = 50000
E = 800000
H = 4
C = 16
D = H * C
EMB = 64
NEG_SLOPE = 0.1

NC = 2
NS = 16
NW = NC * NS

BLK = 128
BLOCKS_PER_W = 200
E_PAD = NW * BLOCKS_PER_W * BLK
NT = 50560
ZROWS = NT // NS
OROWS = N // NS

NBLK = 512
TC_GRID = (N + NBLK - 1) // NBLK


def _prologue_body(x_ref, wg_ref, ad_ref, h_ref, alpha_ref):
    h = jnp.dot(x_ref[...], wg_ref[...], preferred_element_type=jnp.float32)
    hh = h.reshape(NBLK, H, C).transpose(1, 0, 2)
    h_ref[...] = hh
    alpha_ref[...] = jnp.sum(hh * ad_ref[...][:, None, :], axis=-1)


def _epilogue_body(num_ref, den_ref, we_ref, be_ref, out_ref):
    num = num_ref[0] + num_ref[1]
    den = jnp.sum(den_ref[...], axis=1) + 1e-16
    o = (num / den[:, :, None]).transpose(1, 0, 2).reshape(NBLK, D)
    o = jnp.where(o > 0, o, jnp.exp(o) - 1.0)
    out_ref[...] = (
        jnp.dot(o, we_ref[...], preferred_element_type=jnp.float32)
        + be_ref[...]
    )


def _sc_body(h_ref, alpha_ref, asrc_ref, edata_ref, z_ref,
             z1_ref, num_ref, den_ref,
             tab_v, arow_v, rows_v, ebuf_v, p_v,
             num_sp, den_sp, gsem, nsem, dsem, isem):
    cid = lax.axis_index("c")
    sid = lax.axis_index("s")
    wid = cid * NS + sid
    bbase = wid * BLOCKS_PER_W
    iota = lax.iota(jnp.int32, 16)
    zero16 = jnp.zeros((16,), jnp.float32)

    def _head(k, _):
        pltpu.sync_copy(alpha_ref.at[pl.ds(pl.multiple_of(k * NT, 8), NT)],
                        tab_v)
        pltpu.sync_copy(asrc_ref.at[pl.ds(pl.multiple_of(k * C, 8), C)],
                        arow_v)
        pltpu.sync_copy(z_ref, num_sp.at[pl.ds(sid * ZROWS, ZROWS)])
        pltpu.sync_copy(z1_ref, den_sp.at[pl.ds(sid * (NT // NS), NT // NS)])

        arow16 = arow_v[...]
        a_splat = [jnp.full((16,), arow16[c]) for c in range(C)]
        plsc.subcore_barrier()

        def _gather(b, j):
            pltpu.async_copy(h_ref.at[k].at[ebuf_v[j].at[0]], rows_v[j],
                             gsem[j])

        def _wait_scatters(j):
            pltpu.make_async_copy(rows_v[j], num_sp.at[ebuf_v[j].at[1]],
                                  nsem[j]).wait()
            pltpu.make_async_copy(p_v[j], den_sp.at[ebuf_v[j].at[1]],
                                  dsem[j]).wait()

        def _compute(b, j):
            pltpu.make_async_copy(h_ref.at[k].at[ebuf_v[j].at[0]], rows_v[j],
                                  gsem[j]).wait()
            for g in range(BLK // 16):
                ridx = iota + (g * 16)
                cols = []
                alpha = zero16
                for c in range(C):
                    col = plsc.load_gather(
                        rows_v[j], [ridx, jnp.full((16,), c, jnp.int32)])
                    cols.append(col)
                    alpha = alpha + col * a_splat[c]
                dst16 = ebuf_v[j][1, pl.ds(g * 16, 16)]
                w16 = plsc.bitcast(ebuf_v[j][2, pl.ds(g * 16, 16)],
                                   jnp.float32)
                ad = plsc.load_gather(tab_v, [dst16])
                e = alpha + ad
                e = jnp.where(e >= 0.0, e, e * NEG_SLOPE) * w16
                p = jnp.exp(e)
                p_v[j][pl.ds(g * 16, 16)] = p
                for c in range(C):
                    plsc.store_scatter(
                        rows_v[j], [ridx, jnp.full((16,), c, jnp.int32)],
                        cols[c] * p)
            pltpu.async_copy(rows_v[j], num_sp.at[ebuf_v[j].at[1]], nsem[j],
                             add=True)
            pltpu.async_copy(p_v[j], den_sp.at[ebuf_v[j].at[1]], dsem[j],
                             add=True)

        for j in range(4):
            pltpu.sync_copy(edata_ref.at[bbase + j], ebuf_v[j])
        _gather(0, 0)
        _gather(1, 1)
        _compute(0, 0)
        _gather(2, 2)
        _compute(1, 1)
        _wait_scatters(0)
        pltpu.async_copy(edata_ref.at[bbase + 4], ebuf_v[0], isem[0])
        _gather(3, 3)
        _compute(2, 2)
        _wait_scatters(1)
        pltpu.async_copy(edata_ref.at[bbase + 5], ebuf_v[1], isem[1])
        pltpu.make_async_copy(edata_ref.at[bbase + 4], ebuf_v[0],
                              isem[0]).wait()
        _gather(4, 0)
        _compute(3, 3)

        def _quad(t, _):
            for u in range(4):
                b = 4 * t + 4 + u
                j2 = (u + 2) % 4
                j1 = (u + 1) % 4
                _wait_scatters(j2)

                @pl.when(b + 2 < BLOCKS_PER_W)
                def _():
                    pltpu.async_copy(edata_ref.at[bbase + b + 2],
                                     ebuf_v[j2], isem[j2])

                @pl.when(b + 1 < BLOCKS_PER_W)
                def _():
                    pltpu.make_async_copy(
                        edata_ref.at[bbase + b + 1], ebuf_v[j1],
                        isem[j1]).wait()
                    _gather(b + 1, j1)
                _compute(b, u)
            return 0

        lax.fori_loop(0, (BLOCKS_PER_W - 4) // 4, _quad, 0)
        _wait_scatters(2)
        _wait_scatters(3)

        plsc.subcore_barrier()
        pltpu.sync_copy(num_sp.at[pl.ds(sid * ZROWS, ZROWS)],
                        num_ref.at[cid, k, pl.ds(sid * ZROWS, ZROWS)])
        dbase = pl.multiple_of((k * NC + cid) * NT + sid * (NT // NS), 8)
        pltpu.sync_copy(den_sp.at[pl.ds(sid * (NT // NS), NT // NS)],
                        den_ref.at[pl.ds(dbase, NT // NS)])
        plsc.subcore_barrier()
        return 0

    lax.fori_loop(0, H, _head, 0)


_sc_call = pl.kernel(
    _sc_body,
    out_type=(
        jax.ShapeDtypeStruct((NC, H, NT, C), jnp.float32),
        jax.ShapeDtypeStruct((H * NC * NT,), jnp.float32),
    ),
    mesh=plsc.VectorSubcoreMesh(core_axis_name="c", subcore_axis_name="s",
                                num_cores=NC, num_subcores=NS),
    compiler_params=pltpu.CompilerParams(needs_layout_passes=False,
                                         use_tc_tiling_on_sc=False),
    scratch_types=(
        pltpu.VMEM((NT,), jnp.float32),
        pltpu.VMEM((C,), jnp.float32),
        [pltpu.VMEM((BLK, C), jnp.float32)] * 4,
        [pltpu.VMEM((3, BLK), jnp.int32)] * 4,
        [pltpu.VMEM((BLK,), jnp.float32)] * 4,
        pltpu.VMEM_SHARED((NT, C), jnp.float32),
        pltpu.VMEM_SHARED((NT,), jnp.float32),
        [pltpu.SemaphoreType.DMA] * 4,
        [pltpu.SemaphoreType.DMA] * 4,
        [pltpu.SemaphoreType.DMA] * 4,
        [pltpu.SemaphoreType.DMA] * 4,
    ),
)


@functools.partial(jax.jit, static_argnums=())
def kernel(x, edge_index, edge_weights, W_gat, a_src, a_dst, W_emb, b_emb):
    h_heads, alpha_dst = pl.pallas_call(
        _prologue_body,
        grid=(TC_GRID,),
        in_specs=[
            pl.BlockSpec((NBLK, D), lambda i: (i, 0)),
            pl.BlockSpec((D, D), lambda i: (0, 0)),
            pl.BlockSpec((H, C), lambda i: (0, 0)),
        ],
        out_specs=[
            pl.BlockSpec((H, NBLK, C), lambda i: (0, i, 0)),
            pl.BlockSpec((H, NBLK), lambda i: (0, i)),
        ],
        out_shape=[
            jax.ShapeDtypeStruct((H, N, C), jnp.float32),
            jax.ShapeDtypeStruct((H, N), jnp.float32),
        ],
    )(x, W_gat, a_dst)

    alpha_pad = jnp.pad(alpha_dst, ((0, 0), (0, NT - N))).reshape(-1)
    npad = E_PAD - E
    pad_i = lax.iota(jnp.int32, npad)
    src_p = jnp.concatenate([edge_index[0], pad_i % 4096])
    dst_p = jnp.concatenate([edge_index[1], N + pad_i % 512])
    w_p = jnp.concatenate([edge_weights, jnp.zeros((npad,), jnp.float32)])
    edata = jnp.stack([src_p.reshape(-1, BLK), dst_p.reshape(-1, BLK),
                       lax.bitcast_convert_type(w_p, jnp.int32)
                       .reshape(-1, BLK)], axis=1)
    zeros = jnp.zeros((ZROWS, C), jnp.float32)
    zeros1 = jnp.zeros((NT // NS,), jnp.float32)

    num, den = _sc_call(h_heads, alpha_pad, a_src.reshape(-1), edata,
                        zeros, zeros1)

    den4 = den.reshape(H, NC, NT)
    emb = pl.pallas_call(
        _epilogue_body,
        grid=(TC_GRID,),
        in_specs=[
            pl.BlockSpec((NC, H, NBLK, C), lambda i: (0, 0, i, 0)),
            pl.BlockSpec((H, NC, NBLK), lambda i: (0, 0, i)),
            pl.BlockSpec((D, EMB), lambda i: (0, 0)),
            pl.BlockSpec((1, EMB), lambda i: (0, 0)),
        ],
        out_specs=pl.BlockSpec((NBLK, EMB), lambda i: (i, 0)),
        out_shape=jax.ShapeDtypeStruct((N, EMB), jnp.float32),
    )(num, den4, W_emb, b_emb.reshape(1, EMB))
    return emb


# speedup vs baseline: 75.9658x; 1.0833x over previous
import functools

import jax
import jax.numpy as jnp
from jax import lax
from jax.experimental import pallas as pl
from jax.experimental.pallas import tpu as pltpu
from jax.experimental.pallas import tpu_sc as plsc

N = 50000
E = 800000
H = 4
C = 16
D = H * C
EMB = 64
NEG_SLOPE = 0.1

NC = 2
NS = 16
NW = NC * NS

BLK = 128
BLOCKS_PER_W = 200
E_PAD = NW * BLOCKS_PER_W * BLK
NT = 50560
ZROWS = NT // NS
OROWS = N // NS

NBLK = 512
TC_GRID = (N + NBLK - 1) // NBLK
EBLK = 2048
E_GRID = (N + EBLK - 1) // EBLK


def _prologue_body(x_ref, wg_ref, ad_ref, h_ref, alpha_ref):
    h = jnp.dot(x_ref[...], wg_ref[...], preferred_element_type=jnp.float32)
    hh = h.reshape(NBLK, H, C).transpose(1, 0, 2)
    h_ref[...] = hh
    alpha_ref[...] = jnp.sum(hh * ad_ref[...][:, None, :], axis=-1)


def _epilogue_body(num_ref, den_ref, we_ref, be_ref, out_ref):
    num = num_ref[...].reshape(NC, H, EBLK, C)
    den = jnp.sum(den_ref[...], axis=1) + 1e-16
    acc = jnp.broadcast_to(be_ref[...], (EBLK, EMB))
    for k in range(H):
        o = (num[0, k] + num[1, k]) / den[k][:, None]
        o = jnp.where(o > 0, o, jnp.exp(o) - 1.0)
        acc = acc + jnp.dot(o, we_ref[k * C:(k + 1) * C, :],
                            preferred_element_type=jnp.float32)
    out_ref[...] = acc


def _sc_body(h_ref, alpha_ref, asrc_ref, edata_ref, z_ref,
             z1_ref, num_ref, den_ref,
             tab_v, arow_v, rows_v, ebuf_v, p_v,
             num_sp, den_sp, gsem, nsem, dsem, isem):
    cid = lax.axis_index("c")
    sid = lax.axis_index("s")
    wid = cid * NS + sid
    bbase = wid * BLOCKS_PER_W
    iota = lax.iota(jnp.int32, 16)
    zero16 = jnp.zeros((16,), jnp.float32)

    def _head(k, _):
        pltpu.sync_copy(alpha_ref.at[pl.ds(pl.multiple_of(k * NT, 8), NT)],
                        tab_v)
        pltpu.sync_copy(asrc_ref.at[pl.ds(pl.multiple_of(k * C, 8), C)],
                        arow_v)
        pltpu.sync_copy(z_ref, num_sp.at[pl.ds(sid * ZROWS, ZROWS)])
        pltpu.sync_copy(z1_ref, den_sp.at[pl.ds(sid * (NT // NS), NT // NS)])

        arow16 = arow_v[...]
        a_splat = [jnp.full((16,), arow16[c]) for c in range(C)]
        plsc.subcore_barrier()

        def _gather(b, j):
            pltpu.async_copy(h_ref.at[k].at[ebuf_v[j].at[0]], rows_v[j],
                             gsem[j])

        def _wait_scatters(j):
            pltpu.make_async_copy(rows_v[j], num_sp.at[ebuf_v[j].at[1]],
                                  nsem[j]).wait()
            pltpu.make_async_copy(p_v[j], den_sp.at[ebuf_v[j].at[1]],
                                  dsem[j]).wait()

        def _compute(b, j):
            pltpu.make_async_copy(h_ref.at[k].at[ebuf_v[j].at[0]], rows_v[j],
                                  gsem[j]).wait()
            for g in range(BLK // 16):
                ridx = iota + (g * 16)
                cols = []
                alpha = zero16
                for c in range(C):
                    col = plsc.load_gather(
                        rows_v[j], [ridx, jnp.full((16,), c, jnp.int32)])
                    cols.append(col)
                    alpha = alpha + col * a_splat[c]
                dst16 = ebuf_v[j][1, pl.ds(g * 16, 16)]
                w16 = plsc.bitcast(ebuf_v[j][2, pl.ds(g * 16, 16)],
                                   jnp.float32)
                ad = plsc.load_gather(tab_v, [dst16])
                e = alpha + ad
                e = jnp.where(e >= 0.0, e, e * NEG_SLOPE) * w16
                p = jnp.exp(e)
                p_v[j][pl.ds(g * 16, 16)] = p
                for c in range(C):
                    plsc.store_scatter(
                        rows_v[j], [ridx, jnp.full((16,), c, jnp.int32)],
                        cols[c] * p)
            pltpu.async_copy(rows_v[j], num_sp.at[ebuf_v[j].at[1]], nsem[j],
                             add=True)
            pltpu.async_copy(p_v[j], den_sp.at[ebuf_v[j].at[1]], dsem[j],
                             add=True)

        for j in range(4):
            pltpu.sync_copy(edata_ref.at[bbase + j], ebuf_v[j])
        _gather(0, 0)
        _gather(1, 1)
        _compute(0, 0)
        _gather(2, 2)
        _compute(1, 1)
        _wait_scatters(0)
        pltpu.async_copy(edata_ref.at[bbase + 4], ebuf_v[0], isem[0])
        _gather(3, 3)
        _compute(2, 2)
        _wait_scatters(1)
        pltpu.async_copy(edata_ref.at[bbase + 5], ebuf_v[1], isem[1])
        pltpu.make_async_copy(edata_ref.at[bbase + 4], ebuf_v[0],
                              isem[0]).wait()
        _gather(4, 0)
        _compute(3, 3)

        def _quad(t, _):
            for u in range(4):
                b = 4 * t + 4 + u
                j2 = (u + 2) % 4
                j1 = (u + 1) % 4
                _wait_scatters(j2)

                @pl.when(b + 2 < BLOCKS_PER_W)
                def _():
                    pltpu.async_copy(edata_ref.at[bbase + b + 2],
                                     ebuf_v[j2], isem[j2])

                @pl.when(b + 1 < BLOCKS_PER_W)
                def _():
                    pltpu.make_async_copy(
                        edata_ref.at[bbase + b + 1], ebuf_v[j1],
                        isem[j1]).wait()
                    _gather(b + 1, j1)
                _compute(b, u)
            return 0

        lax.fori_loop(0, (BLOCKS_PER_W - 4) // 4, _quad, 0)
        _wait_scatters(2)
        _wait_scatters(3)

        plsc.subcore_barrier()
        pltpu.sync_copy(num_sp.at[pl.ds(sid * ZROWS, ZROWS)],
                        num_ref.at[cid, k, pl.ds(sid * ZROWS, ZROWS)])
        dbase = pl.multiple_of((k * NC + cid) * NT + sid * (NT // NS), 8)
        pltpu.sync_copy(den_sp.at[pl.ds(sid * (NT // NS), NT // NS)],
                        den_ref.at[pl.ds(dbase, NT // NS)])
        plsc.subcore_barrier()
        return 0

    lax.fori_loop(0, H, _head, 0)


_sc_call = pl.kernel(
    _sc_body,
    out_type=(
        jax.ShapeDtypeStruct((NC, H, NT, C), jnp.float32),
        jax.ShapeDtypeStruct((H * NC * NT,), jnp.float32),
    ),
    mesh=plsc.VectorSubcoreMesh(core_axis_name="c", subcore_axis_name="s",
                                num_cores=NC, num_subcores=NS),
    compiler_params=pltpu.CompilerParams(needs_layout_passes=False,
                                         use_tc_tiling_on_sc=False),
    scratch_types=(
        pltpu.VMEM((NT,), jnp.float32),
        pltpu.VMEM((C,), jnp.float32),
        [pltpu.VMEM((BLK, C), jnp.float32)] * 4,
        [pltpu.VMEM((3, BLK), jnp.int32)] * 4,
        [pltpu.VMEM((BLK,), jnp.float32)] * 4,
        pltpu.VMEM_SHARED((NT, C), jnp.float32),
        pltpu.VMEM_SHARED((NT,), jnp.float32),
        [pltpu.SemaphoreType.DMA] * 4,
        [pltpu.SemaphoreType.DMA] * 4,
        [pltpu.SemaphoreType.DMA] * 4,
        [pltpu.SemaphoreType.DMA] * 4,
    ),
)


@functools.partial(jax.jit, static_argnums=())
def kernel(x, edge_index, edge_weights, W_gat, a_src, a_dst, W_emb, b_emb):
    h_heads, alpha_dst = pl.pallas_call(
        _prologue_body,
        grid=(TC_GRID,),
        in_specs=[
            pl.BlockSpec((NBLK, D), lambda i: (i, 0)),
            pl.BlockSpec((D, D), lambda i: (0, 0)),
            pl.BlockSpec((H, C), lambda i: (0, 0)),
        ],
        out_specs=[
            pl.BlockSpec((H, NBLK, C), lambda i: (0, i, 0)),
            pl.BlockSpec((H, NBLK), lambda i: (0, i)),
        ],
        out_shape=[
            jax.ShapeDtypeStruct((H, N, C), jnp.float32),
            jax.ShapeDtypeStruct((H, N), jnp.float32),
        ],
    )(x, W_gat, a_dst)

    alpha_pad = jnp.pad(alpha_dst, ((0, 0), (0, NT - N))).reshape(-1)
    npad = E_PAD - E
    pad_i = lax.iota(jnp.int32, npad)
    src_p = jnp.concatenate([edge_index[0], pad_i % 4096])
    dst_p = jnp.concatenate([edge_index[1], N + pad_i % 512])
    w_p = jnp.concatenate([edge_weights, jnp.zeros((npad,), jnp.float32)])
    edata = jnp.stack([src_p.reshape(-1, BLK), dst_p.reshape(-1, BLK),
                       lax.bitcast_convert_type(w_p, jnp.int32)
                       .reshape(-1, BLK)], axis=1)
    zeros = jnp.zeros((ZROWS, C), jnp.float32)
    zeros1 = jnp.zeros((NT // NS,), jnp.float32)

    num, den = _sc_call(h_heads, alpha_pad, a_src.reshape(-1), edata,
                        zeros, zeros1)

    den4 = den.reshape(H, NC, NT)
    numf = num.reshape(NC, H, NT * C)
    emb = pl.pallas_call(
        _epilogue_body,
        grid=(E_GRID,),
        in_specs=[
            pl.BlockSpec((NC, H, EBLK * C), lambda i: (0, 0, i)),
            pl.BlockSpec((H, NC, EBLK), lambda i: (0, 0, i)),
            pl.BlockSpec((D, EMB), lambda i: (0, 0)),
            pl.BlockSpec((1, EMB), lambda i: (0, 0)),
        ],
        out_specs=pl.BlockSpec((EBLK, EMB), lambda i: (i, 0)),
        out_shape=jax.ShapeDtypeStruct((N, EMB), jnp.float32),
    )(numf, den4, W_emb, b_emb.reshape(1, EMB))
    return emb


# speedup vs baseline: 76.5326x; 1.0075x over previous
import functools

import jax
import jax.numpy as jnp
from jax import lax
from jax.experimental import pallas as pl
from jax.experimental.pallas import tpu as pltpu
from jax.experimental.pallas import tpu_sc as plsc

N = 50000
E = 800000
H = 4
C = 16
D = H * C
EMB = 64
NEG_SLOPE = 0.1

NC = 2
NS = 16
NW = NC * NS

BLK = 128
BLOCKS_PER_W = 200
E_PAD = NW * BLOCKS_PER_W * BLK
NT = 50560
ZROWS = NT // NS
OROWS = N // NS

NBLK = 512
TC_GRID = (N + NBLK - 1) // NBLK
EBLK = 2048
E_GRID = (N + EBLK - 1) // EBLK


def _prologue_body(x_ref, wg_ref, ad_ref, h_ref, alpha_ref):
    h = jnp.dot(x_ref[...], wg_ref[...], preferred_element_type=jnp.float32)
    hh = h.reshape(NBLK, H, C).transpose(1, 0, 2)
    h_ref[...] = hh.reshape(H, NBLK * C)
    alpha_ref[...] = jnp.sum(hh * ad_ref[...][:, None, :], axis=-1)


def _epilogue_body(num_ref, den_ref, we_ref, be_ref, out_ref):
    num = num_ref[...].reshape(NC, H, EBLK, C)
    den = jnp.sum(den_ref[...], axis=1) + 1e-16
    acc = jnp.broadcast_to(be_ref[...], (EBLK, EMB))
    for k in range(H):
        o = (num[0, k] + num[1, k]) / den[k][:, None]
        o = jnp.where(o > 0, o, jnp.exp(o) - 1.0)
        acc = acc + jnp.dot(o, we_ref[k * C:(k + 1) * C, :],
                            preferred_element_type=jnp.float32)
    out_ref[...] = acc


def _sc_body(h_ref, alpha_ref, asrc_ref, edata_ref, z_ref,
             z1_ref, num_ref, den_ref,
             tab_v, arow_v, rows_v, ebuf_v, p_v,
             num_sp, den_sp, gsem, nsem, dsem, isem):
    cid = lax.axis_index("c")
    sid = lax.axis_index("s")
    wid = cid * NS + sid
    bbase = wid * BLOCKS_PER_W
    iota = lax.iota(jnp.int32, 16)
    zero16 = jnp.zeros((16,), jnp.float32)

    def _head(k, _):
        pltpu.sync_copy(alpha_ref.at[pl.ds(pl.multiple_of(k * NT, 8), NT)],
                        tab_v)
        pltpu.sync_copy(asrc_ref.at[pl.ds(pl.multiple_of(k * C, 8), C)],
                        arow_v)
        pltpu.sync_copy(z_ref, num_sp.at[pl.ds(sid * ZROWS, ZROWS)])
        pltpu.sync_copy(z1_ref, den_sp.at[pl.ds(sid * (NT // NS), NT // NS)])

        arow16 = arow_v[...]
        a_splat = [jnp.full((16,), arow16[c]) for c in range(C)]
        plsc.subcore_barrier()

        def _gather(b, j):
            pltpu.async_copy(h_ref.at[k].at[ebuf_v[j].at[0]], rows_v[j],
                             gsem[j])

        def _wait_scatters(j):
            pltpu.make_async_copy(rows_v[j], num_sp.at[ebuf_v[j].at[1]],
                                  nsem[j]).wait()
            pltpu.make_async_copy(p_v[j], den_sp.at[ebuf_v[j].at[1]],
                                  dsem[j]).wait()

        def _compute(b, j):
            pltpu.make_async_copy(h_ref.at[k].at[ebuf_v[j].at[0]], rows_v[j],
                                  gsem[j]).wait()
            for g in range(BLK // 16):
                ridx = iota + (g * 16)
                cols = []
                alpha = zero16
                for c in range(C):
                    col = plsc.load_gather(
                        rows_v[j], [ridx, jnp.full((16,), c, jnp.int32)])
                    cols.append(col)
                    alpha = alpha + col * a_splat[c]
                dst16 = ebuf_v[j][1, pl.ds(g * 16, 16)]
                w16 = plsc.bitcast(ebuf_v[j][2, pl.ds(g * 16, 16)],
                                   jnp.float32)
                ad = plsc.load_gather(tab_v, [dst16])
                e = alpha + ad
                e = jnp.where(e >= 0.0, e, e * NEG_SLOPE) * w16
                p = jnp.exp(e)
                p_v[j][pl.ds(g * 16, 16)] = p
                for c in range(C):
                    plsc.store_scatter(
                        rows_v[j], [ridx, jnp.full((16,), c, jnp.int32)],
                        cols[c] * p)
            pltpu.async_copy(rows_v[j], num_sp.at[ebuf_v[j].at[1]], nsem[j],
                             add=True)
            pltpu.async_copy(p_v[j], den_sp.at[ebuf_v[j].at[1]], dsem[j],
                             add=True)

        for j in range(4):
            pltpu.sync_copy(edata_ref.at[bbase + j], ebuf_v[j])
        _gather(0, 0)
        _gather(1, 1)
        _compute(0, 0)
        _gather(2, 2)
        _compute(1, 1)
        _wait_scatters(0)
        pltpu.async_copy(edata_ref.at[bbase + 4], ebuf_v[0], isem[0])
        _gather(3, 3)
        _compute(2, 2)
        _wait_scatters(1)
        pltpu.async_copy(edata_ref.at[bbase + 5], ebuf_v[1], isem[1])
        pltpu.make_async_copy(edata_ref.at[bbase + 4], ebuf_v[0],
                              isem[0]).wait()
        _gather(4, 0)
        _compute(3, 3)

        def _quad(t, _):
            for u in range(4):
                b = 4 * t + 4 + u
                j2 = (u + 2) % 4
                j1 = (u + 1) % 4
                _wait_scatters(j2)

                @pl.when(b + 2 < BLOCKS_PER_W)
                def _():
                    pltpu.async_copy(edata_ref.at[bbase + b + 2],
                                     ebuf_v[j2], isem[j2])

                @pl.when(b + 1 < BLOCKS_PER_W)
                def _():
                    pltpu.make_async_copy(
                        edata_ref.at[bbase + b + 1], ebuf_v[j1],
                        isem[j1]).wait()
                    _gather(b + 1, j1)
                _compute(b, u)
            return 0

        lax.fori_loop(0, (BLOCKS_PER_W - 4) // 4, _quad, 0)
        _wait_scatters(2)
        _wait_scatters(3)

        plsc.subcore_barrier()
        pltpu.sync_copy(num_sp.at[pl.ds(sid * ZROWS, ZROWS)],
                        num_ref.at[cid, k, pl.ds(sid * ZROWS, ZROWS)])
        dbase = pl.multiple_of((k * NC + cid) * NT + sid * (NT // NS), 8)
        pltpu.sync_copy(den_sp.at[pl.ds(sid * (NT // NS), NT // NS)],
                        den_ref.at[pl.ds(dbase, NT // NS)])
        plsc.subcore_barrier()
        return 0

    lax.fori_loop(0, H, _head, 0)


_sc_call = pl.kernel(
    _sc_body,
    out_type=(
        jax.ShapeDtypeStruct((NC, H, NT, C), jnp.float32),
        jax.ShapeDtypeStruct((H * NC * NT,), jnp.float32),
    ),
    mesh=plsc.VectorSubcoreMesh(core_axis_name="c", subcore_axis_name="s",
                                num_cores=NC, num_subcores=NS),
    compiler_params=pltpu.CompilerParams(needs_layout_passes=False,
                                         use_tc_tiling_on_sc=False),
    scratch_types=(
        pltpu.VMEM((NT,), jnp.float32),
        pltpu.VMEM((C,), jnp.float32),
        [pltpu.VMEM((BLK, C), jnp.float32)] * 4,
        [pltpu.VMEM((3, BLK), jnp.int32)] * 4,
        [pltpu.VMEM((BLK,), jnp.float32)] * 4,
        pltpu.VMEM_SHARED((NT, C), jnp.float32),
        pltpu.VMEM_SHARED((NT,), jnp.float32),
        [pltpu.SemaphoreType.DMA] * 4,
        [pltpu.SemaphoreType.DMA] * 4,
        [pltpu.SemaphoreType.DMA] * 4,
        [pltpu.SemaphoreType.DMA] * 4,
    ),
)


@functools.partial(jax.jit, static_argnums=())
def kernel(x, edge_index, edge_weights, W_gat, a_src, a_dst, W_emb, b_emb):
    h_heads, alpha_dst = pl.pallas_call(
        _prologue_body,
        grid=(TC_GRID,),
        in_specs=[
            pl.BlockSpec((NBLK, D), lambda i: (i, 0)),
            pl.BlockSpec((D, D), lambda i: (0, 0)),
            pl.BlockSpec((H, C), lambda i: (0, 0)),
        ],
        out_specs=[
            pl.BlockSpec((H, NBLK * C), lambda i: (0, i)),
            pl.BlockSpec((H, NBLK), lambda i: (0, i)),
        ],
        out_shape=[
            jax.ShapeDtypeStruct((H, N * C), jnp.float32),
            jax.ShapeDtypeStruct((H, N), jnp.float32),
        ],
    )(x, W_gat, a_dst)
    h_heads = h_heads.reshape(H, N, C)

    alpha_pad = jnp.pad(alpha_dst, ((0, 0), (0, NT - N))).reshape(-1)
    npad = E_PAD - E
    pad_i = lax.iota(jnp.int32, npad)
    src_p = jnp.concatenate([edge_index[0], pad_i % 4096])
    dst_p = jnp.concatenate([edge_index[1], N + pad_i % 512])
    w_p = jnp.concatenate([edge_weights, jnp.zeros((npad,), jnp.float32)])
    edata = jnp.stack([src_p.reshape(-1, BLK), dst_p.reshape(-1, BLK),
                       lax.bitcast_convert_type(w_p, jnp.int32)
                       .reshape(-1, BLK)], axis=1)
    zeros = jnp.zeros((ZROWS, C), jnp.float32)
    zeros1 = jnp.zeros((NT // NS,), jnp.float32)

    num, den = _sc_call(h_heads, alpha_pad, a_src.reshape(-1), edata,
                        zeros, zeros1)

    den4 = den.reshape(H, NC, NT)
    numf = num.reshape(NC, H, NT * C)
    emb = pl.pallas_call(
        _epilogue_body,
        grid=(E_GRID,),
        in_specs=[
            pl.BlockSpec((NC, H, EBLK * C), lambda i: (0, 0, i)),
            pl.BlockSpec((H, NC, EBLK), lambda i: (0, 0, i)),
            pl.BlockSpec((D, EMB), lambda i: (0, 0)),
            pl.BlockSpec((1, EMB), lambda i: (0, 0)),
        ],
        out_specs=pl.BlockSpec((EBLK, EMB), lambda i: (i, 0)),
        out_shape=jax.ShapeDtypeStruct((N, EMB), jnp.float32),
    )(numf, den4, W_emb, b_emb.reshape(1, EMB))
    return emb


# speedup vs baseline: 77.4297x; 1.0117x over previous
import functools

import jax
import jax.numpy as jnp
from jax import lax
from jax.experimental import pallas as pl
from jax.experimental.pallas import tpu as pltpu
from jax.experimental.pallas import tpu_sc as plsc

N = 50000
E = 800000
H = 4
C = 16
D = H * C
EMB = 64
NEG_SLOPE = 0.1

NC = 2
NS = 16
NW = NC * NS

BLK = 128
BLOCKS_PER_W = 200
E_PAD = NW * BLOCKS_PER_W * BLK
NT = 50560
ZROWS = NT // NS
OROWS = N // NS

NBLK = 512
TC_GRID = (N + NBLK - 1) // NBLK
EBLK = 2048
E_GRID = (N + EBLK - 1) // EBLK


def _prologue_body(x_ref, wg_ref, ad_ref, h_ref, alpha_ref):
    h = jnp.dot(x_ref[...], wg_ref[...], preferred_element_type=jnp.float32)
    hh = h.reshape(NBLK, H, C).transpose(1, 0, 2)
    h_ref[...] = hh.reshape(H, NBLK * C)
    alpha_ref[...] = jnp.sum(hh * ad_ref[...][:, None, :], axis=-1)


def _epilogue_body(num_ref, den_ref, we_ref, be_ref, out_ref):
    num = num_ref[...].reshape(NC, H, EBLK, C)
    den = jnp.sum(den_ref[...], axis=1) + 1e-16
    acc = jnp.broadcast_to(be_ref[...], (EBLK, EMB))
    for k in range(H):
        o = (num[0, k] + num[1, k]) / den[k][:, None]
        o = jnp.where(o > 0, o, jnp.exp(o) - 1.0)
        acc = acc + jnp.dot(o, we_ref[k * C:(k + 1) * C, :],
                            preferred_element_type=jnp.float32)
    out_ref[...] = acc


def _sc_body(h_ref, alpha_ref, asrc_ref, edata_ref, z_ref,
             z1_ref, num_ref, den_ref,
             tab_v, arow_v, rows_v, ebuf_v, p_v,
             num_sp, den_sp, gsem, nsem, dsem, isem):
    cid = lax.axis_index("c")
    sid = lax.axis_index("s")
    wid = cid * NS + sid
    bbase = wid * BLOCKS_PER_W
    iota = lax.iota(jnp.int32, 16)
    zero16 = jnp.zeros((16,), jnp.float32)

    def _head(k, _):
        d1 = pltpu.async_copy(
            alpha_ref.at[pl.ds(pl.multiple_of(k * NT, 8), NT)], tab_v,
            gsem[0])
        d2 = pltpu.async_copy(
            asrc_ref.at[pl.ds(pl.multiple_of(k * C, 8), C)], arow_v,
            gsem[1])
        d3 = pltpu.async_copy(z_ref, num_sp.at[pl.ds(sid * ZROWS, ZROWS)],
                              gsem[2])
        d4 = pltpu.async_copy(
            z1_ref, den_sp.at[pl.ds(sid * (NT // NS), NT // NS)], gsem[3])
        d1.wait()
        d2.wait()
        d3.wait()
        d4.wait()

        arow16 = arow_v[...]
        a_splat = [jnp.full((16,), arow16[c]) for c in range(C)]
        plsc.subcore_barrier()

        def _gather(b, j):
            pltpu.async_copy(h_ref.at[k].at[ebuf_v[j].at[0]], rows_v[j],
                             gsem[j])

        def _wait_scatters(j):
            pltpu.make_async_copy(rows_v[j], num_sp.at[ebuf_v[j].at[1]],
                                  nsem[j]).wait()
            pltpu.make_async_copy(p_v[j], den_sp.at[ebuf_v[j].at[1]],
                                  dsem[j]).wait()

        def _compute(b, j):
            pltpu.make_async_copy(h_ref.at[k].at[ebuf_v[j].at[0]], rows_v[j],
                                  gsem[j]).wait()
            for g in range(BLK // 16):
                ridx = iota + (g * 16)
                cols = []
                alpha = zero16
                for c in range(C):
                    col = plsc.load_gather(
                        rows_v[j], [ridx, jnp.full((16,), c, jnp.int32)])
                    cols.append(col)
                    alpha = alpha + col * a_splat[c]
                dst16 = ebuf_v[j][1, pl.ds(g * 16, 16)]
                w16 = plsc.bitcast(ebuf_v[j][2, pl.ds(g * 16, 16)],
                                   jnp.float32)
                ad = plsc.load_gather(tab_v, [dst16])
                e = alpha + ad
                e = jnp.where(e >= 0.0, e, e * NEG_SLOPE) * w16
                p = jnp.exp(e)
                p_v[j][pl.ds(g * 16, 16)] = p
                for c in range(C):
                    plsc.store_scatter(
                        rows_v[j], [ridx, jnp.full((16,), c, jnp.int32)],
                        cols[c] * p)
            pltpu.async_copy(rows_v[j], num_sp.at[ebuf_v[j].at[1]], nsem[j],
                             add=True)
            pltpu.async_copy(p_v[j], den_sp.at[ebuf_v[j].at[1]], dsem[j],
                             add=True)

        for j in range(4):
            pltpu.sync_copy(edata_ref.at[bbase + j], ebuf_v[j])
        _gather(0, 0)
        _gather(1, 1)
        _compute(0, 0)
        _gather(2, 2)
        _compute(1, 1)
        _wait_scatters(0)
        pltpu.async_copy(edata_ref.at[bbase + 4], ebuf_v[0], isem[0])
        _gather(3, 3)
        _compute(2, 2)
        _wait_scatters(1)
        pltpu.async_copy(edata_ref.at[bbase + 5], ebuf_v[1], isem[1])
        pltpu.make_async_copy(edata_ref.at[bbase + 4], ebuf_v[0],
                              isem[0]).wait()
        _gather(4, 0)
        _compute(3, 3)

        def _quad(t, _):
            for u in range(4):
                b = 4 * t + 4 + u
                j2 = (u + 2) % 4
                j1 = (u + 1) % 4
                _wait_scatters(j2)

                @pl.when(b + 2 < BLOCKS_PER_W)
                def _():
                    pltpu.async_copy(edata_ref.at[bbase + b + 2],
                                     ebuf_v[j2], isem[j2])

                @pl.when(b + 1 < BLOCKS_PER_W)
                def _():
                    pltpu.make_async_copy(
                        edata_ref.at[bbase + b + 1], ebuf_v[j1],
                        isem[j1]).wait()
                    _gather(b + 1, j1)
                _compute(b, u)
            return 0

        lax.fori_loop(0, (BLOCKS_PER_W - 4) // 4, _quad, 0)
        _wait_scatters(2)
        _wait_scatters(3)

        plsc.subcore_barrier()
        dbase = pl.multiple_of((k * NC + cid) * NT + sid * (NT // NS), 8)
        o1 = pltpu.async_copy(num_sp.at[pl.ds(sid * ZROWS, ZROWS)],
                              num_ref.at[cid, k, pl.ds(sid * ZROWS, ZROWS)],
                              gsem[0])
        o2 = pltpu.async_copy(den_sp.at[pl.ds(sid * (NT // NS), NT // NS)],
                              den_ref.at[pl.ds(dbase, NT // NS)], gsem[1])
        o1.wait()
        o2.wait()
        plsc.subcore_barrier()
        return 0

    lax.fori_loop(0, H, _head, 0)


_sc_call = pl.kernel(
    _sc_body,
    out_type=(
        jax.ShapeDtypeStruct((NC, H, NT, C), jnp.float32),
        jax.ShapeDtypeStruct((H * NC * NT,), jnp.float32),
    ),
    mesh=plsc.VectorSubcoreMesh(core_axis_name="c", subcore_axis_name="s",
                                num_cores=NC, num_subcores=NS),
    compiler_params=pltpu.CompilerParams(needs_layout_passes=False,
                                         use_tc_tiling_on_sc=False),
    scratch_types=(
        pltpu.VMEM((NT,), jnp.float32),
        pltpu.VMEM((C,), jnp.float32),
        [pltpu.VMEM((BLK, C), jnp.float32)] * 4,
        [pltpu.VMEM((3, BLK), jnp.int32)] * 4,
        [pltpu.VMEM((BLK,), jnp.float32)] * 4,
        pltpu.VMEM_SHARED((NT, C), jnp.float32),
        pltpu.VMEM_SHARED((NT,), jnp.float32),
        [pltpu.SemaphoreType.DMA] * 4,
        [pltpu.SemaphoreType.DMA] * 4,
        [pltpu.SemaphoreType.DMA] * 4,
        [pltpu.SemaphoreType.DMA] * 4,
    ),
)


@functools.partial(jax.jit, static_argnums=())
def kernel(x, edge_index, edge_weights, W_gat, a_src, a_dst, W_emb, b_emb):
    h_heads, alpha_dst = pl.pallas_call(
        _prologue_body,
        grid=(TC_GRID,),
        in_specs=[
            pl.BlockSpec((NBLK, D), lambda i: (i, 0)),
            pl.BlockSpec((D, D), lambda i: (0, 0)),
            pl.BlockSpec((H, C), lambda i: (0, 0)),
        ],
        out_specs=[
            pl.BlockSpec((H, NBLK * C), lambda i: (0, i)),
            pl.BlockSpec((H, NBLK), lambda i: (0, i)),
        ],
        out_shape=[
            jax.ShapeDtypeStruct((H, N * C), jnp.float32),
            jax.ShapeDtypeStruct((H, N), jnp.float32),
        ],
    )(x, W_gat, a_dst)
    h_heads = h_heads.reshape(H, N, C)

    alpha_pad = jnp.pad(alpha_dst, ((0, 0), (0, NT - N))).reshape(-1)
    npad = E_PAD - E
    pad_i = lax.iota(jnp.int32, npad)
    src_p = jnp.concatenate([edge_index[0], pad_i % 4096])
    dst_p = jnp.concatenate([edge_index[1], N + pad_i % 512])
    w_p = jnp.concatenate([edge_weights, jnp.zeros((npad,), jnp.float32)])
    edata = jnp.stack([src_p.reshape(-1, BLK), dst_p.reshape(-1, BLK),
                       lax.bitcast_convert_type(w_p, jnp.int32)
                       .reshape(-1, BLK)], axis=1)
    zeros = jnp.zeros((ZROWS, C), jnp.float32)
    zeros1 = jnp.zeros((NT // NS,), jnp.float32)

    num, den = _sc_call(h_heads, alpha_pad, a_src.reshape(-1), edata,
                        zeros, zeros1)

    den4 = den.reshape(H, NC, NT)
    numf = num.reshape(NC, H, NT * C)
    emb = pl.pallas_call(
        _epilogue_body,
        grid=(E_GRID,),
        in_specs=[
            pl.BlockSpec((NC, H, EBLK * C), lambda i: (0, 0, i)),
            pl.BlockSpec((H, NC, EBLK), lambda i: (0, 0, i)),
            pl.BlockSpec((D, EMB), lambda i: (0, 0)),
            pl.BlockSpec((1, EMB), lambda i: (0, 0)),
        ],
        out_specs=pl.BlockSpec((EBLK, EMB), lambda i: (i, 0)),
        out_shape=jax.ShapeDtypeStruct((N, EMB), jnp.float32),
    )(numf, den4, W_emb, b_emb.reshape(1, EMB))
    return emb
